# baseline (speedup 1.0000x reference)
import functools

import jax
import jax.numpy as jnp
from jax import lax
from jax.experimental import pallas as pl
from jax.experimental.pallas import tpu as pltpu
from jax.experimental.pallas import tpu_sc as plsc

B = 16384
L = 200
D = 64
V = 1000000
NC, NS, LANES = 2, 16, 16
NW = NC * NS
ROWS_PER_W = B // NW
INV_L = 1.0 / L
KV = D // LANES

RCH = 800
NRCH = V // RCH
VR = V // 4

G = 2
CHUNKS = ROWS_PER_W // G
TOK = G * L
IDXW = 50
NGATH = TOK // IDXW
STAGE = 8


def _repack(table):
    mesh = plsc.VectorSubcoreMesh(
        core_axis_name="c", subcore_axis_name="s",
        num_cores=NC, num_subcores=NS)

    @functools.partial(
        pl.kernel,
        out_type=jax.ShapeDtypeStruct((VR, 128), jnp.int32),
        mesh=mesh,
        compiler_params=pltpu.CompilerParams(needs_layout_passes=False),
        scratch_types=[
            pltpu.VMEM((RCH, D), jnp.float32),
            pltpu.VMEM((RCH // 4, 128), jnp.int32),
        ],
    )
    def repack(table_ref, out_ref, in_v, out_v):
        wid = lax.axis_index("s") * NC + lax.axis_index("c")

        def step(t, carry):
            c = wid + t * NW

            @pl.when(c < NRCH)
            def _():
                src0 = pl.multiple_of(c * RCH, 8)
                pltpu.sync_copy(table_ref.at[pl.ds(src0, RCH)], in_v)

                def body(i, carry2):
                    for u in range(4):
                        r = i * 4 + u
                        vs = [in_v[r, pl.ds(k * LANES, LANES)]
                              for k in range(KV)]
                        h0 = plsc.pack(vs[0], vs[1],
                                       format=plsc.PackFormat.INTERLEAVED)
                        h1 = plsc.pack(vs[2], vs[3],
                                       format=plsc.PackFormat.INTERLEAVED)
                        out_v[i, pl.ds(u * 32, LANES)] = plsc.bitcast(
                            h0, jnp.int32)
                        out_v[i, pl.ds(u * 32 + LANES, LANES)] = plsc.bitcast(
                            h1, jnp.int32)
                    return carry2

                lax.fori_loop(0, RCH // 4, body, 0, unroll=2)
                dst0 = pl.multiple_of(c * (RCH // 4), 8)
                pltpu.sync_copy(out_v, out_ref.at[pl.ds(dst0, RCH // 4)])
            return carry

        lax.fori_loop(0, NRCH // NW + 1, step, 0)

    return repack(table)


def _issue_gathers(table_ref, idx_ref, rows_ref, sem):
    for j in range(NGATH):
        pltpu.async_copy(
            table_ref.at[idx_ref.at[j]],
            rows_ref.at[pl.ds(j * IDXW, IDXW)],
            sem,
        )


def _drain(table_ref, rows_ref, sem):
    pltpu.make_async_copy(table_ref.at[pl.ds(0, TOK)], rows_ref, sem).wait()


def _token_group(rows_ref, offv, r0, nlanes, accs):
    a = list(accs)
    for u in range(nlanes):
        r = r0 + u
        off = offv[u]
        w0 = rows_ref[r, pl.ds(off, LANES)]
        w1 = rows_ref[r, pl.ds(off + LANES, LANES)]
        e0, o0 = plsc.unpack(plsc.bitcast(w0, jnp.bfloat16),
                             format=plsc.PackFormat.INTERLEAVED,
                             preferred_element_type=jnp.float32)
        e1, o1 = plsc.unpack(plsc.bitcast(w1, jnp.bfloat16),
                             format=plsc.PackFormat.INTERLEAVED,
                             preferred_element_type=jnp.float32)
        a[0] = a[0] + e0
        a[1] = a[1] + o0
        a[2] = a[2] + e1
        a[3] = a[3] + o1
    return tuple(a)


def _accumulate(rows_ref, xf_ref, out_stage, srow):
    for g in range(G):
        base = g * L
        zero = jnp.zeros((LANES,), jnp.float32)

        def body(i, accs, base=base):
            r0 = base + i * LANES
            offv = (xf_ref[pl.ds(r0, LANES)] & 3) * 32
            return _token_group(rows_ref, offv, r0, LANES, accs)

        accs = lax.fori_loop(0, L // LANES, body, (zero,) * KV)
        tail = L % LANES
        if tail:
            r0 = base + (L // LANES) * LANES
            offv = (xf_ref[pl.ds(r0, LANES)] & 3) * 32
            accs = _token_group(rows_ref, offv, r0, tail, accs)
        for k in range(KV):
            m = jnp.maximum(accs[k] * INV_L, 0.0)
            out_stage[srow + g, pl.ds(k * LANES, LANES)] = m


def _sc_bow(xd2, xflat, packed):
    mesh = plsc.VectorSubcoreMesh(
        core_axis_name="c", subcore_axis_name="s",
        num_cores=NC, num_subcores=NS)

    @functools.partial(
        pl.kernel,
        out_type=jax.ShapeDtypeStruct((B, D), jnp.float32),
        mesh=mesh,
        compiler_params=pltpu.CompilerParams(needs_layout_passes=False),
        scratch_types=[
            pltpu.VMEM((NGATH, IDXW), jnp.int32),
            pltpu.VMEM((NGATH, IDXW), jnp.int32),
            pltpu.VMEM((TOK + LANES,), jnp.int32),
            pltpu.VMEM((TOK + LANES,), jnp.int32),
            pltpu.VMEM((TOK, 128), jnp.int32),
            pltpu.VMEM((TOK, 128), jnp.int32),
            pltpu.VMEM((STAGE, D), jnp.float32),
            pltpu.SemaphoreType.DMA,
            pltpu.SemaphoreType.DMA,
        ],
    )
    def bow(xd2_ref, xf_ref, table_ref, out_ref,
            idx0, idx1, xf0, xf1, rows0, rows1, out_stage, sem0, sem1):
        wid = lax.axis_index("s") * NC + lax.axis_index("c")
        xrow0 = wid * (CHUNKS * NGATH)
        tok0 = wid * (CHUNKS * TOK)
        orow0 = wid * ROWS_PER_W

        def stage_in(c, idxb, xfb, rowsb, sem):
            i0 = pl.multiple_of(xrow0 + c * NGATH, 8)
            pltpu.sync_copy(xd2_ref.at[pl.ds(i0, NGATH)], idxb)
            f0 = pl.multiple_of(tok0 + c * TOK, 8)
            pltpu.sync_copy(xf_ref.at[pl.ds(f0, TOK)], xfb.at[pl.ds(0, TOK)])
            _issue_gathers(table_ref, idxb, rowsb, sem)

        stage_in(0, idx0, xf0, rows0, sem0)

        def step(t, carry):
            stage_in(2 * t + 1, idx1, xf1, rows1, sem1)
            _drain(table_ref, rows0, sem0)
            _accumulate(rows0, xf0, out_stage, (t % 2) * 2 * G)

            @pl.when(t < CHUNKS // 2 - 1)
            def _():
                stage_in(2 * t + 2, idx0, xf0, rows0, sem0)

            _drain(table_ref, rows1, sem1)
            _accumulate(rows1, xf1, out_stage, (t % 2) * 2 * G + G)

            @pl.when(t % 2 == 1)
            def _():
                o0 = pl.multiple_of(orow0 + (t - 1) * 2 * G, 8)
                pltpu.sync_copy(out_stage, out_ref.at[pl.ds(o0, STAGE)])
            return carry

        lax.fori_loop(0, CHUNKS // 2, step, 0)

    return bow(xd2, xflat, packed)


def _tc_classify(m, wt, b2):
    def body(m_ref, w_ref, b_ref, o_ref):
        o_ref[...] = (
            jnp.dot(m_ref[...], w_ref[...], preferred_element_type=jnp.float32)
            + b_ref[...])

    grid = 16
    bm = B // grid
    return pl.pallas_call(
        body,
        grid=(grid,),
        in_specs=[
            pl.BlockSpec((bm, D), lambda i: (i, 0)),
            pl.BlockSpec((D, 8), lambda i: (0, 0)),
            pl.BlockSpec((1, 8), lambda i: (0, 0)),
        ],
        out_specs=pl.BlockSpec((bm, 8), lambda i: (i, 0)),
        out_shape=jax.ShapeDtypeStruct((B, 8), jnp.float32),
    )(m, wt, b2)


def kernel(x, emb_table, W, b):
    xi = x.astype(jnp.int32)
    xd2 = (xi >> 2).reshape(-1, IDXW)
    xflat = xi.reshape(-1)
    packed = _repack(emb_table)
    pooled = _sc_bow(xd2, xflat, packed)
    nc = W.shape[0]
    wt = jnp.zeros((D, 8), jnp.float32).at[:, :nc].set(W.T)
    b2 = jnp.zeros((1, 8), jnp.float32).at[0, :nc].set(b)
    logits = _tc_classify(pooled, wt, b2)
    return logits[:, :nc]


# speedup vs baseline: 1.4663x; 1.4663x over previous
import functools

import jax
import jax.numpy as jnp
from jax import lax
from jax.experimental import pallas as pl
from jax.experimental.pallas import tpu as pltpu
from jax.experimental.pallas import tpu_sc as plsc

B = 16384
L = 200
D = 64
NC, NS, LANES = 2, 16, 16
NW = NC * NS
ROWS_PER_W = B // NW
G = 4
CHUNKS = ROWS_PER_W // G
TOK = G * L
IDXW = 100
NGATH = TOK // IDXW
X2W = 100
KV = D // LANES
INV_L = 1.0 / L


def _issue_gathers(table_ref, idx_ref, rows_ref, sem):
    for j in range(NGATH):
        pltpu.async_copy(
            table_ref.at[idx_ref.at[j]],
            rows_ref.at[pl.ds(j * IDXW, IDXW)],
            sem,
        )


def _drain(table_ref, rows_ref, sem):
    pltpu.make_async_copy(table_ref.at[pl.ds(0, TOK)], rows_ref, sem).wait()


def _accumulate(rows_ref, out_stage, slot):
    for g in range(G):
        base = g * L
        zero = jnp.zeros((LANES,), jnp.float32)

        def body(i, accs, base=base):
            a = list(accs)
            for u in range(4):
                r = base + i * 4 + u
                for k in range(KV):
                    a[k] = a[k] + rows_ref[r, pl.ds(k * LANES, LANES)]
            return tuple(a)

        accs = lax.fori_loop(0, L // 4, body, (zero,) * KV, unroll=2)
        for k in range(KV):
            m = jnp.maximum(accs[k] * INV_L, 0.0)
            out_stage[slot * G + g, pl.ds(k * LANES, LANES)] = m


def _sc_bow(x2, table):
    mesh = plsc.VectorSubcoreMesh(
        core_axis_name="c", subcore_axis_name="s",
        num_cores=NC, num_subcores=NS)

    @functools.partial(
        pl.kernel,
        out_type=jax.ShapeDtypeStruct((B, D), jnp.float32),
        mesh=mesh,
        compiler_params=pltpu.CompilerParams(
            use_tc_tiling_on_sc=False, needs_layout_passes=False),
        scratch_types=[
            pltpu.VMEM((NGATH, IDXW), jnp.int32),
            pltpu.VMEM((NGATH, IDXW), jnp.int32),
            pltpu.VMEM((TOK, D), jnp.float32),
            pltpu.VMEM((TOK, D), jnp.float32),
            pltpu.VMEM((2 * G, D), jnp.float32),
            pltpu.SemaphoreType.DMA,
            pltpu.SemaphoreType.DMA,
        ],
    )
    def bow(x2_ref, table_ref, out_ref,
            idx0, idx1, rows0, rows1, out_stage, sem0, sem1):
        wid = lax.axis_index("s") * NC + lax.axis_index("c")
        xrow0 = wid * (CHUNKS * NGATH)
        orow0 = wid * ROWS_PER_W

        pltpu.sync_copy(x2_ref.at[pl.ds(xrow0, NGATH)], idx0)
        _issue_gathers(table_ref, idx0, rows0, sem0)

        def step(t, carry):
            pltpu.sync_copy(
                x2_ref.at[pl.ds(xrow0 + (2 * t + 1) * NGATH, NGATH)], idx1)
            _issue_gathers(table_ref, idx1, rows1, sem1)
            _drain(table_ref, rows0, sem0)
            _accumulate(rows0, out_stage, 0)

            @pl.when(t < CHUNKS // 2 - 1)
            def _():
                pltpu.sync_copy(
                    x2_ref.at[pl.ds(xrow0 + (2 * t + 2) * NGATH, NGATH)], idx0)
                _issue_gathers(table_ref, idx0, rows0, sem0)

            _drain(table_ref, rows1, sem1)
            _accumulate(rows1, out_stage, 1)

            pltpu.sync_copy(out_stage,
                            out_ref.at[pl.ds(orow0 + t * (2 * G), 2 * G)])
            return carry

        lax.fori_loop(0, CHUNKS // 2, step, 0)

    return bow(x2, table)


def _tc_classify(m, wt, b2):
    def body(m_ref, w_ref, b_ref, o_ref):
        o_ref[...] = (
            jnp.dot(m_ref[...], w_ref[...], preferred_element_type=jnp.float32)
            + b_ref[...])

    grid = 16
    bm = B // grid
    return pl.pallas_call(
        body,
        grid=(grid,),
        in_specs=[
            pl.BlockSpec((bm, D), lambda i: (i, 0)),
            pl.BlockSpec((D, 8), lambda i: (0, 0)),
            pl.BlockSpec((1, 8), lambda i: (0, 0)),
        ],
        out_specs=pl.BlockSpec((bm, 8), lambda i: (i, 0)),
        out_shape=jax.ShapeDtypeStruct((B, 8), jnp.float32),
    )(m, wt, b2)


def kernel(x, emb_table, W, b):
    x2 = x.astype(jnp.int32).reshape(-1, X2W)
    pooled = _sc_bow(x2, emb_table)
    nc = W.shape[0]
    wt = jnp.zeros((D, 8), jnp.float32).at[:, :nc].set(W.T)
    b2 = jnp.zeros((1, 8), jnp.float32).at[0, :nc].set(b)
    logits = _tc_classify(pooled, wt, b2)
    return logits[:, :nc]


# speedup vs baseline: 1.4709x; 1.0031x over previous
import functools

import jax
import jax.numpy as jnp
from jax import lax
from jax.experimental import pallas as pl
from jax.experimental.pallas import tpu as pltpu
from jax.experimental.pallas import tpu_sc as plsc

B = 16384
L = 200
D = 64
NC, NS, LANES = 2, 16, 16
NW = NC * NS
ROWS_PER_W = B // NW
G = 4
CHUNKS = ROWS_PER_W // G
TOK = G * L
IDXW = 100
NGATH = TOK // IDXW
X2W = 100
KV = D // LANES
INV_L = 1.0 / L


def _issue_gathers(table_ref, idx_ref, rows_ref, sem):
    for j in range(NGATH):
        pltpu.async_copy(
            table_ref.at[idx_ref.at[j]],
            rows_ref.at[pl.ds(j * IDXW, IDXW)],
            sem,
        )


def _drain(table_ref, rows_ref, sem):
    pltpu.make_async_copy(table_ref.at[pl.ds(0, TOK)], rows_ref, sem).wait()


def _accumulate(rows_ref, out_stage, slot):
    for g in range(G):
        base = g * L
        zero = jnp.zeros((LANES,), jnp.float32)

        def body(i, accs, base=base):
            a = list(accs)
            for u in range(4):
                r = base + i * 4 + u
                for k in range(KV):
                    a[k] = a[k] + rows_ref[r, pl.ds(k * LANES, LANES)]
            return tuple(a)

        accs = lax.fori_loop(0, L // 4, body, (zero,) * KV, unroll=2)
        for k in range(KV):
            m = jnp.maximum(accs[k] * INV_L, 0.0)
            out_stage[slot * G + g, pl.ds(k * LANES, LANES)] = m


def _sc_bow(x2, table):
    mesh = plsc.VectorSubcoreMesh(
        core_axis_name="c", subcore_axis_name="s",
        num_cores=NC, num_subcores=NS)

    @functools.partial(
        pl.kernel,
        out_type=jax.ShapeDtypeStruct((B, D), jnp.float32),
        mesh=mesh,
        compiler_params=pltpu.CompilerParams(use_tc_tiling_on_sc=False),
        scratch_types=[
            pltpu.VMEM((NGATH, IDXW), jnp.int32),
            pltpu.VMEM((NGATH, IDXW), jnp.int32),
            pltpu.VMEM((TOK, D), jnp.float32),
            pltpu.VMEM((TOK, D), jnp.float32),
            pltpu.VMEM((2 * G, D), jnp.float32),
            pltpu.SemaphoreType.DMA,
            pltpu.SemaphoreType.DMA,
        ],
    )
    def bow(x2_ref, table_ref, out_ref,
            idx0, idx1, rows0, rows1, out_stage, sem0, sem1):
        wid = lax.axis_index("s") * NC + lax.axis_index("c")
        xrow0 = wid * (CHUNKS * NGATH)
        orow0 = wid * ROWS_PER_W

        pltpu.sync_copy(x2_ref.at[pl.ds(xrow0, NGATH)], idx0)
        _issue_gathers(table_ref, idx0, rows0, sem0)

        def step(t, carry):
            pltpu.sync_copy(
                x2_ref.at[pl.ds(xrow0 + (2 * t + 1) * NGATH, NGATH)], idx1)
            _issue_gathers(table_ref, idx1, rows1, sem1)
            _drain(table_ref, rows0, sem0)
            _accumulate(rows0, out_stage, 0)

            @pl.when(t < CHUNKS // 2 - 1)
            def _():
                pltpu.sync_copy(
                    x2_ref.at[pl.ds(xrow0 + (2 * t + 2) * NGATH, NGATH)], idx0)
                _issue_gathers(table_ref, idx0, rows0, sem0)

            _drain(table_ref, rows1, sem1)
            _accumulate(rows1, out_stage, 1)

            pltpu.sync_copy(out_stage,
                            out_ref.at[pl.ds(orow0 + t * (2 * G), 2 * G)])
            return carry

        lax.fori_loop(0, CHUNKS // 2, step, 0)

    return bow(x2, table)


def _tc_classify(m, wt, b2):
    def body(m_ref, w_ref, b_ref, o_ref):
        o_ref[...] = (
            jnp.dot(m_ref[...], w_ref[...], preferred_element_type=jnp.float32)
            + b_ref[...])

    grid = 16
    bm = B // grid
    return pl.pallas_call(
        body,
        grid=(grid,),
        in_specs=[
            pl.BlockSpec((bm, D), lambda i: (i, 0)),
            pl.BlockSpec((D, 8), lambda i: (0, 0)),
            pl.BlockSpec((1, 8), lambda i: (0, 0)),
        ],
        out_specs=pl.BlockSpec((bm, 8), lambda i: (i, 0)),
        out_shape=jax.ShapeDtypeStruct((B, 8), jnp.float32),
    )(m, wt, b2)


def kernel(x, emb_table, W, b):
    x2 = x.astype(jnp.int32).reshape(-1, X2W)
    pooled = _sc_bow(x2, emb_table)
    nc = W.shape[0]
    wt = jnp.zeros((D, 8), jnp.float32).at[:, :nc].set(W.T)
    b2 = jnp.zeros((1, 8), jnp.float32).at[0, :nc].set(b)
    logits = _tc_classify(pooled, wt, b2)
    return logits[:, :nc]
